# baseline (speedup 1.0000x reference)
import functools

import jax
import jax.numpy as jnp
from jax import lax
from jax.experimental import pallas as pl
from jax.experimental.pallas import tpu as pltpu
from jax.experimental.pallas import tpu_sc as plsc

N = 10000
D = 128
E = 320000
NC = 2
NS = 16
NW = NC * NS
CB = 128
NCH = 80
EWP = NCH * CB
NBUF = 2
NGRP = NCH // NBUF
NP = 10240
RPS = NP // NS
CR = NP // D
RB = 1024
NB = NP // RB


def _segment_sum_sc(feat, edges, zeros_init, with_cnt):
    mesh = plsc.VectorSubcoreMesh(core_axis_name="c", subcore_axis_name="s")
    out_type = [jax.ShapeDtypeStruct((NC, NP, D), jnp.float32)]
    scratch = [
        pltpu.VMEM((NBUF, 2, CB), jnp.int32),
        pltpu.VMEM((NBUF, CB, D), jnp.float32),
        pltpu.VMEM_SHARED((NP, D), jnp.float32),
        pltpu.SemaphoreType.DMA((NBUF,)),
        pltpu.SemaphoreType.DMA((NBUF,)),
    ]
    if with_cnt:
        out_type.append(jax.ShapeDtypeStruct((NC, CR, D), jnp.float32))
        scratch += [
            pltpu.VMEM((CR, D), jnp.float32),
            pltpu.VMEM((CR,), jnp.int32),
            pltpu.VMEM_SHARED((CR, D), jnp.float32),
        ]

    def seg_kernel(feat_h, edge_h, zz_h, *rest):
        if with_cnt:
            (sum_h, cnt_h, idx_v, rows_v, acc, isems, gsems, cntl_v, iid_v,
             cnts) = rest
        else:
            sum_h, idx_v, rows_v, acc, isems, gsems = rest
        c = lax.axis_index("c")
        s = lax.axis_index("s")
        w = s * NC + c

        def _wait_idx(b):
            pltpu.make_async_copy(edge_h.at[w, pl.ds(0, 2)], idx_v.at[b],
                                  isems.at[b]).wait()

        def _wait_rows(b):
            pltpu.make_async_copy(feat_h.at[pl.ds(0, CB)], rows_v.at[b],
                                  gsems.at[b]).wait()

        for b in range(NBUF):
            pltpu.async_copy(edge_h.at[w, pl.ds(2 * b, 2)], idx_v.at[b],
                             isems.at[b])
        for b in range(NBUF):
            _wait_idx(b)
            pltpu.async_copy(feat_h.at[idx_v.at[b, 0]], rows_v.at[b],
                             gsems.at[b])
        pltpu.sync_copy(zz_h.at[pl.ds(s * RPS, RPS)],
                        acc.at[pl.ds(s * RPS, RPS)])
        if with_cnt:
            for k in range(CR // 16):
                iid_v[pl.ds(16 * k, 16)] = lax.iota(jnp.int32, 16) + 16 * k
            pltpu.sync_copy(zz_h.at[pl.ds(0, CR)], cntl_v)

            @pl.when(s == 0)
            def _():
                pltpu.sync_copy(zz_h.at[pl.ds(0, CR)], cnts)
        plsc.subcore_barrier()

        ones = jnp.ones((16,), jnp.float32)

        def _count(b):
            for k in range(CB // 16):
                dv = idx_v[b, 1, pl.ds(16 * k, 16)]
                row = lax.shift_right_logical(dv, 7)
                col = lax.bitwise_and(dv, 127)
                plsc.addupdate_scatter(cntl_v, [row, col], ones)

        def _step(i, b):
            _wait_rows(b)
            pltpu.sync_copy(rows_v.at[b], acc.at[idx_v.at[b, 1]], add=True)
            if with_cnt:
                _count(b)

        def group(g, carry):
            for b in range(NBUF):
                i = g * NBUF + b
                _step(i, b)
                pltpu.async_copy(edge_h.at[w, pl.ds(2 * (i + NBUF), 2)],
                                 idx_v.at[b], isems.at[b])
                _wait_idx(b)
                pltpu.async_copy(feat_h.at[idx_v.at[b, 0]], rows_v.at[b],
                                 gsems.at[b])
            return carry

        lax.fori_loop(0, NGRP - 1, group, 0)
        for b in range(NBUF):
            _step((NGRP - 1) * NBUF + b, b)
        if with_cnt:
            plsc.subcore_barrier()
            pltpu.sync_copy(cntl_v, cnts.at[iid_v], add=True)
        plsc.subcore_barrier()
        pltpu.sync_copy(acc.at[pl.ds(s * RPS, RPS)],
                        sum_h.at[c, pl.ds(s * RPS, RPS)])
        if with_cnt:
            @pl.when(s == 0)
            def _():
                pltpu.sync_copy(cnts, cnt_h.at[c])

    f = pl.kernel(
        seg_kernel,
        out_type=tuple(out_type) if with_cnt else out_type[0],
        mesh=mesh,
        compiler_params=pltpu.CompilerParams(use_tc_tiling_on_sc=False,
                                             needs_layout_passes=False),
        scratch_types=scratch,
    )
    return f(feat, edges, zeros_init)


def _dotT(a, w):
    return lax.dot_general(a, w, (((1,), (1,)), ((), ())),
                           preferred_element_type=jnp.float32)


def _layer1_body(agg_ref, cnt_ref, x_ref, wl_ref, bl_ref, wr_ref, o_ref):
    ssum = agg_ref[0] + agg_ref[1]
    mean = ssum / cnt_ref[...]
    y = _dotT(mean, wl_ref[...]) + bl_ref[...] + _dotT(x_ref[...], wr_ref[...])
    o_ref[...] = jnp.maximum(y, 0.0)


def _layer2_body(agg_ref, cnt_ref, y1_ref, wl_ref, bl_ref, wr_ref, wm_ref,
                 bm_ref, o_ref):
    ssum = agg_ref[0] + agg_ref[1]
    mean = ssum / cnt_ref[...]
    t = _dotT(mean, wl_ref[...]) + bl_ref[...] + _dotT(y1_ref[...],
                                                       wr_ref[...])
    t = jnp.maximum(t, 0.0)
    o_ref[...] = _dotT(t, wm_ref[...]) + bm_ref[...]


def _wspec():
    return pl.BlockSpec((D, D), lambda i: (0, 0))


def _bspec():
    return pl.BlockSpec((1, D), lambda i: (0, 0))


_AGG_SPEC = pl.BlockSpec((NC, RB, D), lambda i: (0, i, 0))
_ROW_SPEC = pl.BlockSpec((RB, D), lambda i: (i, 0))
_CNT_SPEC = pl.BlockSpec((RB, 1), lambda i: (i, 0))


def _layer1_tc(agg, cntcol, x, W_l, b_l, W_r):
    return pl.pallas_call(
        _layer1_body,
        grid=(NB,),
        in_specs=[_AGG_SPEC, _CNT_SPEC, _ROW_SPEC, _wspec(), _bspec(),
                  _wspec()],
        out_specs=_ROW_SPEC,
        out_shape=jax.ShapeDtypeStruct((N, D), jnp.float32),
    )(agg, cntcol, x, W_l, b_l.reshape(1, D), W_r)


def _layer2_tc(agg, cntcol, y1, W_l, b_l, W_r, Wm, bm):
    return pl.pallas_call(
        _layer2_body,
        grid=(NB,),
        in_specs=[_AGG_SPEC, _CNT_SPEC, _ROW_SPEC, _wspec(), _bspec(),
                  _wspec(), _wspec(), _bspec()],
        out_specs=_ROW_SPEC,
        out_shape=jax.ShapeDtypeStruct((N, D), jnp.float32),
    )(agg, cntcol, y1, W_l, b_l.reshape(1, D), W_r, Wm, bm.reshape(1, D))


@jax.jit
def kernel(x, edge_index, batch, W1_l, b1_l, W1_r, W2_l, b2_l, W2_r, Wm, bm):
    del batch
    src = edge_index[0].astype(jnp.int32).reshape(NW, E // NW)
    dst = edge_index[1].astype(jnp.int32).reshape(NW, E // NW)
    pad = EWP - E // NW
    src = jnp.pad(src, ((0, 0), (0, pad))).reshape(NW, NCH, CB)
    dst = jnp.pad(dst, ((0, 0), (0, pad)), constant_values=NP - 1)
    dst = dst.reshape(NW, NCH, CB)
    edges = jnp.stack([src, dst], axis=2).reshape(NW, 2 * NCH, CB)
    zeros_init = jnp.zeros((NP, D), jnp.float32)

    agg1, cnt = _segment_sum_sc(x, edges, zeros_init, True)
    cntcol = jnp.maximum((cnt[0] + cnt[1]).reshape(NP, 1), 1.0)
    y1 = _layer1_tc(agg1, cntcol, x, W1_l, b1_l, W1_r)
    agg2 = _segment_sum_sc(y1, edges, zeros_init, False)
    return _layer2_tc(agg2, cntcol, y1, W2_l, b2_l, W2_r, Wm, bm)


# speedup vs baseline: 1.4221x; 1.4221x over previous
import functools

import jax
import jax.numpy as jnp
from jax import lax
from jax.experimental import pallas as pl
from jax.experimental.pallas import tpu as pltpu
from jax.experimental.pallas import tpu_sc as plsc

N = 10000
D = 128
DP = 144
E = 320000
NC = 2
NS = 16
NW = NC * NS
EW = E // NW
CB = 40
NCH = 252
NSL = 6
NIS = 12
NSGR = NCH // NIS
NP = 10240
RPS = NP // NS
RB = 1000
NB = N // RB


def _segment_sum_sc(feat, edges4, zeros_init):
    mesh = plsc.VectorSubcoreMesh(core_axis_name="c", subcore_axis_name="s")

    @functools.partial(
        pl.kernel,
        out_type=jax.ShapeDtypeStruct((NC, NP, DP), jnp.float32),
        mesh=mesh,
        compiler_params=pltpu.CompilerParams(use_tc_tiling_on_sc=False),
        scratch_types=[
            pltpu.VMEM((NIS, 2, CB), jnp.int32),
            pltpu.VMEM((NSL, CB, DP), jnp.float32),
            pltpu.VMEM_SHARED((NP, DP), jnp.float32),
            pltpu.SemaphoreType.DMA((NIS,)),
            pltpu.SemaphoreType.DMA((NSL,)),
            pltpu.SemaphoreType.DMA((NSL,)),
        ],
    )
    def seg_kernel(feat_h, edge_h, zz_h, out_h, idx_v, rows_v, acc, isems,
                   gsems, ssems):
        c = lax.axis_index("c")
        s = lax.axis_index("s")
        w = s * NC + c

        def _wait_idx(r):
            pltpu.make_async_copy(edge_h.at[w, 0], idx_v.at[r],
                                  isems.at[r]).wait()

        def _wait_rows(b):
            pltpu.make_async_copy(feat_h.at[pl.ds(0, CB)], rows_v.at[b],
                                  gsems.at[b]).wait()

        def _wait_scat(b, r):
            pltpu.make_async_copy(rows_v.at[b], acc.at[idx_v.at[r, 1]],
                                  ssems.at[b]).wait()

        def _fire_idx(r, chunk):
            pltpu.async_copy(edge_h.at[w, chunk], idx_v.at[r], isems.at[r])

        def _fire_gather(b, r):
            pltpu.async_copy(feat_h.at[idx_v.at[r, 0]], rows_v.at[b],
                             gsems.at[b])

        for r in range(6):
            _fire_idx(r, r)
        for q in range(3):
            _wait_idx(q)
            _fire_gather(q, q)
        pltpu.sync_copy(zz_h.at[pl.ds(s * RPS, RPS)],
                        acc.at[pl.ds(s * RPS, RPS)])
        plsc.subcore_barrier()

        def supergroup(g, first, last):
            base = NIS * g
            for k in range(NIS):
                b = k % NSL
                bg = (k + 3) % NSL
                _wait_rows(b)
                pltpu.async_copy(rows_v.at[b], acc.at[idx_v.at[k, 1]],
                                 ssems.at[b], add=True)
                if not (first and k < 3):
                    _wait_scat(bg, (k + 9) % NIS)
                if not (last and k >= 6):
                    _fire_idx((k + 6) % NIS, base + k + 6)
                if not (last and k >= 9):
                    _wait_idx((k + 3) % NIS)
                    _fire_gather(bg, (k + 3) % NIS)

        supergroup(0, True, False)
        lax.fori_loop(1, NSGR - 1,
                      lambda g, cy: (supergroup(g, False, False), cy)[1], 0)
        supergroup(NSGR - 1, False, True)
        for k in range(9, 12):
            _wait_scat(k % NSL, k)
        plsc.subcore_barrier()
        pltpu.sync_copy(acc.at[pl.ds(s * RPS, RPS)],
                        out_h.at[c, pl.ds(s * RPS, RPS)])

    return seg_kernel(feat, edges4, zeros_init)


def _dotT(a, w):
    return lax.dot_general(a, w, (((1,), (1,)), ((), ())),
                           preferred_element_type=jnp.float32)


def _layer1_body(agg_ref, x_ref, wl_ref, bl_ref, wr_ref, o_ref):
    ssum = agg_ref[0] + agg_ref[1]
    cnt = jnp.maximum(ssum[:, D:D + 1], 1.0)
    mean = ssum[:, :D] / cnt
    y = _dotT(mean, wl_ref[...]) + bl_ref[...] + _dotT(x_ref[...], wr_ref[...])
    y = jnp.maximum(y, 0.0)
    o_ref[...] = jnp.concatenate(
        [y, jnp.ones((RB, 1), jnp.float32), jnp.zeros((RB, 15), jnp.float32)],
        axis=1)


def _layer2_body(agg_ref, y1_ref, wl_ref, bl_ref, wr_ref, wm_ref, bm_ref,
                 o_ref):
    ssum = agg_ref[0] + agg_ref[1]
    cnt = jnp.maximum(ssum[:, D:D + 1], 1.0)
    mean = ssum[:, :D] / cnt
    y1 = y1_ref[:, :D]
    t = _dotT(mean, wl_ref[...]) + bl_ref[...] + _dotT(y1, wr_ref[...])
    t = jnp.maximum(t, 0.0)
    o_ref[...] = _dotT(t, wm_ref[...]) + bm_ref[...]


def _wspec():
    return pl.BlockSpec((D, D), lambda i: (0, 0))


def _bspec():
    return pl.BlockSpec((1, D), lambda i: (0, 0))


_AGG_SPEC = pl.BlockSpec((NC, RB, DP), lambda i: (0, i, 0))


def _layer1_tc(agg, x, W_l, b_l, W_r):
    return pl.pallas_call(
        _layer1_body,
        grid=(NB,),
        in_specs=[
            _AGG_SPEC,
            pl.BlockSpec((RB, D), lambda i: (i, 0)),
            _wspec(), _bspec(), _wspec(),
        ],
        out_specs=pl.BlockSpec((RB, DP), lambda i: (i, 0)),
        out_shape=jax.ShapeDtypeStruct((N, DP), jnp.float32),
    )(agg, x, W_l, b_l.reshape(1, D), W_r)


def _layer2_tc(agg, y1a, W_l, b_l, W_r, Wm, bm):
    return pl.pallas_call(
        _layer2_body,
        grid=(NB,),
        in_specs=[
            _AGG_SPEC,
            pl.BlockSpec((RB, DP), lambda i: (i, 0)),
            _wspec(), _bspec(), _wspec(), _wspec(), _bspec(),
        ],
        out_specs=pl.BlockSpec((RB, D), lambda i: (i, 0)),
        out_shape=jax.ShapeDtypeStruct((N, D), jnp.float32),
    )(agg, y1a, W_l, b_l.reshape(1, D), W_r, Wm, bm.reshape(1, D))


@jax.jit
def kernel(x, edge_index, batch, W1_l, b1_l, W1_r, W2_l, b2_l, W2_r, Wm, bm):
    del batch
    src = edge_index[0].astype(jnp.int32).reshape(NW, EW)
    dst = edge_index[1].astype(jnp.int32).reshape(NW, EW)
    pad = NCH * CB - EW
    src3 = jnp.pad(src, ((0, 0), (0, pad))).reshape(NW, NCH, CB)
    dst3 = jnp.pad(dst, ((0, 0), (0, pad)),
                   constant_values=NP - 1).reshape(NW, NCH, CB)
    edges4 = jnp.stack([src3, dst3], axis=2)
    zeros_init = jnp.zeros((NP, DP), jnp.float32)
    xa = jnp.concatenate(
        [x, jnp.ones((N, 1), jnp.float32), jnp.zeros((N, 15), jnp.float32)],
        axis=1)

    agg1 = _segment_sum_sc(xa, edges4, zeros_init)
    y1a = _layer1_tc(agg1, x, W1_l, b1_l, W1_r)
    agg2 = _segment_sum_sc(y1a, edges4, zeros_init)
    return _layer2_tc(agg2, y1a, W2_l, b2_l, W2_r, Wm, bm)


# speedup vs baseline: 1.6387x; 1.1523x over previous
import functools

import jax
import jax.numpy as jnp
from jax import lax
from jax.experimental import pallas as pl
from jax.experimental.pallas import tpu as pltpu
from jax.experimental.pallas import tpu_sc as plsc

N = 10000
D = 128
DP = 144
E = 320000
NC = 2
NS = 16
NW = NC * NS
EW = E // NW
CB = 40
NCH = EW // CB
NBUF = 5
NGRP = NCH // NBUF
NP = 10240
RPS = NP // NS
RB = 1000
NB = N // RB


def _segment_sum_sc(feat, edges4, zeros_init, dp):
    mesh = plsc.VectorSubcoreMesh(core_axis_name="c", subcore_axis_name="s")

    @functools.partial(
        pl.kernel,
        out_type=jax.ShapeDtypeStruct((NC, NP, dp), jnp.float32),
        mesh=mesh,
        compiler_params=pltpu.CompilerParams(use_tc_tiling_on_sc=False),
        scratch_types=[
            pltpu.VMEM((NBUF, 2, CB), jnp.int32),
            pltpu.VMEM((NBUF, CB, dp), jnp.float32),
            pltpu.VMEM_SHARED((NP, dp), jnp.float32),
            pltpu.SemaphoreType.DMA((NBUF,)),
            pltpu.SemaphoreType.DMA((NBUF,)),
        ],
    )
    def seg_kernel(feat_h, edge_h, zz_h, out_h, idx_v, rows_v, acc, isems,
                   gsems):
        c = lax.axis_index("c")
        s = lax.axis_index("s")
        w = s * NC + c

        def _wait_idx(b):
            pltpu.make_async_copy(edge_h.at[w, 0], idx_v.at[b],
                                  isems.at[b]).wait()

        def _wait_rows(b):
            pltpu.make_async_copy(feat_h.at[pl.ds(0, CB)], rows_v.at[b],
                                  gsems.at[b]).wait()

        for b in range(NBUF):
            pltpu.async_copy(edge_h.at[w, b], idx_v.at[b], isems.at[b])
        for b in range(NBUF):
            _wait_idx(b)
            pltpu.async_copy(feat_h.at[idx_v.at[b, 0]], rows_v.at[b],
                             gsems.at[b])
        pltpu.sync_copy(zz_h.at[pl.ds(s * RPS, RPS)],
                        acc.at[pl.ds(s * RPS, RPS)])
        plsc.subcore_barrier()

        def group(g, carry):
            for b in range(NBUF):
                i = g * NBUF + b
                _wait_rows(b)
                pltpu.sync_copy(rows_v.at[b], acc.at[idx_v.at[b, 1]],
                                add=True)
                pltpu.async_copy(edge_h.at[w, i + NBUF], idx_v.at[b],
                                 isems.at[b])
                _wait_idx(b)
                pltpu.async_copy(feat_h.at[idx_v.at[b, 0]], rows_v.at[b],
                                 gsems.at[b])
            return carry

        lax.fori_loop(0, NGRP - 1, group, 0)
        for b in range(NBUF):
            _wait_rows(b)
            pltpu.sync_copy(rows_v.at[b], acc.at[idx_v.at[b, 1]], add=True)
        plsc.subcore_barrier()
        pltpu.sync_copy(acc.at[pl.ds(s * RPS, RPS)],
                        out_h.at[c, pl.ds(s * RPS, RPS)])

    return seg_kernel(feat, edges4, zeros_init)


def _dotT(a, w):
    return lax.dot_general(a, w, (((1,), (1,)), ((), ())),
                           preferred_element_type=jnp.float32)


def _layer1_body(agg_ref, x_ref, wl_ref, bl_ref, wr_ref, o_ref, c_ref):
    ssum = agg_ref[0] + agg_ref[1]
    cnt = jnp.maximum(ssum[:, D:D + 1], 1.0)
    mean = ssum[:, :D] / cnt
    y = _dotT(mean, wl_ref[...]) + bl_ref[...] + _dotT(x_ref[...], wr_ref[...])
    o_ref[...] = jnp.maximum(y, 0.0)
    c_ref[...] = cnt


def _layer2_body(agg_ref, cnt_ref, y1_ref, wl_ref, bl_ref, wr_ref, wm_ref,
                 bm_ref, o_ref):
    ssum = agg_ref[0] + agg_ref[1]
    mean = ssum / cnt_ref[...]
    t = _dotT(mean, wl_ref[...]) + bl_ref[...] + _dotT(y1_ref[...],
                                                       wr_ref[...])
    t = jnp.maximum(t, 0.0)
    o_ref[...] = _dotT(t, wm_ref[...]) + bm_ref[...]


def _wspec():
    return pl.BlockSpec((D, D), lambda i: (0, 0))


def _bspec():
    return pl.BlockSpec((1, D), lambda i: (0, 0))


_AGG1_SPEC = pl.BlockSpec((NC, RB, DP), lambda i: (0, i, 0))
_AGG2_SPEC = pl.BlockSpec((NC, RB, D), lambda i: (0, i, 0))
_ROW_SPEC = pl.BlockSpec((RB, D), lambda i: (i, 0))
_CNT_SPEC = pl.BlockSpec((RB, 1), lambda i: (i, 0))


def _layer1_tc(agg, x, W_l, b_l, W_r):
    return pl.pallas_call(
        _layer1_body,
        grid=(NB,),
        in_specs=[
            _AGG1_SPEC, _ROW_SPEC,
            _wspec(), _bspec(), _wspec(),
        ],
        out_specs=[_ROW_SPEC, _CNT_SPEC],
        out_shape=[jax.ShapeDtypeStruct((N, D), jnp.float32),
                   jax.ShapeDtypeStruct((N, 1), jnp.float32)],
    )(agg, x, W_l, b_l.reshape(1, D), W_r)


def _layer2_tc(agg, cntc, y1, W_l, b_l, W_r, Wm, bm):
    return pl.pallas_call(
        _layer2_body,
        grid=(NB,),
        in_specs=[
            _AGG2_SPEC, _CNT_SPEC, _ROW_SPEC,
            _wspec(), _bspec(), _wspec(), _wspec(), _bspec(),
        ],
        out_specs=_ROW_SPEC,
        out_shape=jax.ShapeDtypeStruct((N, D), jnp.float32),
    )(agg, cntc, y1, W_l, b_l.reshape(1, D), W_r, Wm, bm.reshape(1, D))


@jax.jit
def kernel(x, edge_index, batch, W1_l, b1_l, W1_r, W2_l, b2_l, W2_r, Wm, bm):
    del batch
    src3 = edge_index[0].astype(jnp.int32).reshape(NW, NCH, CB)
    dst3 = edge_index[1].astype(jnp.int32).reshape(NW, NCH, CB)
    edges4 = jnp.stack([src3, dst3], axis=2)
    xa = jnp.concatenate(
        [x, jnp.ones((N, 1), jnp.float32), jnp.zeros((N, 15), jnp.float32)],
        axis=1)

    agg1 = _segment_sum_sc(xa, edges4, jnp.zeros((NP, DP), jnp.float32), DP)
    y1, cntc = _layer1_tc(agg1, x, W1_l, b1_l, W1_r)
    agg2 = _segment_sum_sc(y1, edges4, jnp.zeros((NP, D), jnp.float32), D)
    return _layer2_tc(agg2, cntc, y1, W2_l, b2_l, W2_r, Wm, bm)
